# baseline (speedup 1.0000x reference)
import functools

import jax
import jax.numpy as jnp
from jax import lax
from jax.experimental import pallas as pl
from jax.experimental.pallas import tpu as pltpu
from jax.experimental.pallas import tpu_sc as plsc

_B = 16384
_D = 64
_L = 16

_RB = 512
_NTB = (1000000 + _RB - 1) // _RB


def _tc_transpose_body(t_ref, out_ref):
    blk = t_ref[...]
    tr = jnp.transpose(blk, (1, 0))
    r3 = tr.reshape(_RB // 2, 2, 64)
    out_ref[...] = jnp.concatenate([r3[:, 0, :], r3[:, 1, :]], axis=1)


def _tc_transpose(table_t):
    return pl.pallas_call(
        _tc_transpose_body,
        grid=(_NTB,),
        in_specs=[pl.BlockSpec((64, _RB), lambda i: (0, i))],
        out_specs=pl.BlockSpec((_RB // 2, 2 * _D), lambda i: (i, 0)),
        out_shape=jax.ShapeDtypeStruct((1000000 // 2, 2 * _D), jnp.float32),
    )(table_t)


_info = plsc.get_sparse_core_info()
_NC = _info.num_cores
_NS = _info.num_subcores
_NW = _NC * _NS
_BW = _B // _NW
_Q = 128
_NQ = _BW // _Q


def _sc_body(pidx_u_hbm, pidx_i_hbm, poff_u_hbm, poff_i_hbm, meta_hbm,
             ut_hbm, it_hbm, cw_hbm, mw_hbm, bias_hbm, out_hbm,
             idx_u, idx_i, poff_u, poff_i, ru0, ru1, ri0, ri1, meta_v,
             cw_v, mw_v, bias_v, out_v, sem0, sem1):
    wid = lax.axis_index("s") * _NC + lax.axis_index("c")

    pltpu.sync_copy(pidx_u_hbm.at[wid], idx_u)
    pltpu.sync_copy(pidx_i_hbm.at[wid], idx_i)
    pltpu.sync_copy(poff_u_hbm.at[wid], poff_u)
    pltpu.sync_copy(poff_i_hbm.at[wid], poff_i)
    pltpu.sync_copy(cw_hbm, cw_v)
    pltpu.sync_copy(mw_hbm, mw_v)
    pltpu.sync_copy(bias_hbm, bias_v)
    meta_h = pltpu.async_copy(meta_hbm.at[wid], meta_v, sem0)

    rubufs = (ru0, ru1)
    ribufs = (ri0, ri1)
    sems = (sem0, sem1)

    def fire(q):
        s = sems[q % 2]
        return (pltpu.async_copy(ut_hbm.at[idx_u.at[q]], rubufs[q % 2], s),
                pltpu.async_copy(it_hbm.at[idx_i.at[q]], ribufs[q % 2], s))

    cw_regs = [cw_v[pl.ds(c * _L, _L)] for c in range(_D // _L)]
    mw_regs = [mw_v[pl.ds(c * _L, _L)] for c in range(_D // _L)]
    bias_vec = bias_v[...]
    lane = lax.iota(jnp.int32, 16)

    pending = fire(0)
    meta_h.wait()

    for q in range(_NQ):
        nxt = fire(q + 1) if q + 1 < _NQ else None
        for h in pending:
            h.wait()
        pending = nxt
        ru = rubufs[q % 2]
        ri = ribufs[q % 2]

        def group(g, carry, q=q, ru=ru, ri=ri):
            res = jnp.zeros((_L,), jnp.float32)
            pv_u = poff_u[pl.ds(q * _Q + g * _L, _L)]
            pv_i = poff_i[pl.ds(q * _Q + g * _L, _L)]
            for r in range(_L):
                slot = g * _L + r
                pu = pv_u[r]
                pi = pv_i[r]
                mrow = q * (_Q // 2) + g * (_L // 2) + (r // 2)
                mcol = (r % 2) * _D
                acc = None
                for c in range(_D // _L):
                    u = ru[slot, pl.ds(pu + c * _L, _L)]
                    it = ri[slot, pl.ds(pi + c * _L, _L)]
                    m = meta_v[mrow, pl.ds(mcol + c * _L, _L)]
                    t = u * it * cw_regs[c] + m * mw_regs[c]
                    acc = t if acc is None else acc + t
                tot = jnp.sum(acc)
                res = jnp.where(lane == r, tot, res)
            out_v[pl.ds(q * _Q + g * _L, _L)] = res + bias_vec
            return carry

        lax.fori_loop(0, _Q // _L, group, 0)

    pltpu.sync_copy(out_v, out_hbm.at[pl.ds(wid * _BW, _BW)])


@jax.jit
def _run(pidx_u, pidx_i, poff_u, poff_i, meta, ut, it, cw, mw, bias16):
    mesh = plsc.VectorSubcoreMesh(core_axis_name="c", subcore_axis_name="s")
    f = functools.partial(
        pl.kernel,
        mesh=mesh,
        compiler_params=pltpu.CompilerParams(
            needs_layout_passes=False, use_tc_tiling_on_sc=True),
        out_type=jax.ShapeDtypeStruct((_B,), jnp.float32),
        scratch_types=[
            pltpu.VMEM((_NQ, _Q), jnp.int32),
            pltpu.VMEM((_NQ, _Q), jnp.int32),
            pltpu.VMEM((_BW,), jnp.int32),
            pltpu.VMEM((_BW,), jnp.int32),
            pltpu.VMEM((_Q, 2 * _D), jnp.float32),
            pltpu.VMEM((_Q, 2 * _D), jnp.float32),
            pltpu.VMEM((_Q, 2 * _D), jnp.float32),
            pltpu.VMEM((_Q, 2 * _D), jnp.float32),
            pltpu.VMEM((_BW // 2, 2 * _D), jnp.float32),
            pltpu.VMEM((_D,), jnp.float32),
            pltpu.VMEM((_D,), jnp.float32),
            pltpu.VMEM((_L,), jnp.float32),
            pltpu.VMEM((_BW,), jnp.float32),
            pltpu.SemaphoreType.DMA,
            pltpu.SemaphoreType.DMA,
        ],
    )(_sc_body)
    return f(pidx_u, pidx_i, poff_u, poff_i, meta, ut, it, cw, mw, bias16)


def kernel(user, item, item_metadata, user_table, item_table, comb_w, comb_b,
           meta_w, meta_b, global_bias):
    user = user.astype(jnp.int32)
    item = item.astype(jnp.int32)
    ut = _tc_transpose(user_table.T)
    it = _tc_transpose(item_table.T)
    pidx_u = (user >> 1).reshape(_NW, _NQ, _Q)
    pidx_i = (item >> 1).reshape(_NW, _NQ, _Q)
    poff_u = ((user & 1) * _D).reshape(_NW, _BW)
    poff_i = ((item & 1) * _D).reshape(_NW, _BW)
    meta = item_metadata.reshape(_NW, _BW // 2, 2 * _D)
    cw = comb_w.reshape(_D)
    mw = meta_w.reshape(_D)
    bias16 = jnp.broadcast_to(comb_b + meta_b + global_bias, (_L,)).astype(jnp.float32)
    return _run(pidx_u, pidx_i, poff_u, poff_i, meta, ut, it, cw, mw, bias16)


# speedup vs baseline: 4.0249x; 4.0249x over previous
import functools

import jax
import jax.numpy as jnp
from jax import lax
from jax.experimental import pallas as pl
from jax.experimental.pallas import tpu as pltpu
from jax.experimental.pallas import tpu_sc as plsc

_B = 16384
_D = 64
_L = 16

_RB = 2048
_NTB = 245
_OFF = 244 * _RB
_NPR = _NTB * _RB


def _tc_transpose_body(t1_ref, t2_ref, out_ref):
    eye = jnp.eye(128, dtype=jnp.bfloat16)
    b1 = t1_ref[...].astype(jnp.bfloat16)
    b2 = t2_ref[...].astype(jnp.bfloat16)
    for m in range(_RB // 128):
        sl = slice(m * 128, (m + 1) * 128)
        tr1 = jax.lax.dot_general(
            eye, b1[:, sl], (((0,), (1,)), ((), ())),
            preferred_element_type=jnp.float32)
        tr2 = jax.lax.dot_general(
            eye, b2[:, sl], (((0,), (1,)), ((), ())),
            preferred_element_type=jnp.float32)
        out_ref[pl.ds(m * 128, 128), :] = jnp.concatenate([tr1, tr2], axis=1)


def _tc_transpose(table_t):
    return pl.pallas_call(
        _tc_transpose_body,
        grid=(_NTB,),
        in_specs=[pl.BlockSpec((64, _RB), lambda i: (0, i)),
                  pl.BlockSpec((64, _RB), lambda i: (0, i + _OFF // _RB))],
        out_specs=pl.BlockSpec((_RB, 2 * _D), lambda i: (i, 0)),
        out_shape=jax.ShapeDtypeStruct((_NPR, 2 * _D), jnp.float32),
    )(table_t, table_t)


_info = plsc.get_sparse_core_info()
_NC = _info.num_cores
_NS = _info.num_subcores
_NW = _NC * _NS
_BW = _B // _NW
_Q = 128
_NQ = _BW // _Q


def _sc_body(pidx_u_hbm, pidx_i_hbm, poff_u_hbm, poff_i_hbm, meta_hbm,
             ut_hbm, it_hbm, cw_hbm, mw_hbm, bias_hbm, out_hbm,
             idx_u, idx_i, poff_u, poff_i, ru0, ru1, ri0, ri1, meta_v,
             cw_v, mw_v, bias_v, out_v, sem0, sem1):
    wid = lax.axis_index("s") * _NC + lax.axis_index("c")

    pltpu.sync_copy(pidx_u_hbm.at[wid], idx_u)
    pltpu.sync_copy(pidx_i_hbm.at[wid], idx_i)
    pltpu.sync_copy(poff_u_hbm.at[wid], poff_u)
    pltpu.sync_copy(poff_i_hbm.at[wid], poff_i)
    pltpu.sync_copy(cw_hbm, cw_v)
    pltpu.sync_copy(mw_hbm, mw_v)
    pltpu.sync_copy(bias_hbm, bias_v)
    meta_h = pltpu.async_copy(meta_hbm.at[wid], meta_v, sem0)

    rubufs = (ru0, ru1)
    ribufs = (ri0, ri1)
    sems = (sem0, sem1)

    def fire(q):
        s = sems[q % 2]
        return (pltpu.async_copy(ut_hbm.at[idx_u.at[q]], rubufs[q % 2], s),
                pltpu.async_copy(it_hbm.at[idx_i.at[q]], ribufs[q % 2], s))

    cw_regs = [cw_v[pl.ds(c * _L, _L)] for c in range(_D // _L)]
    mw_regs = [mw_v[pl.ds(c * _L, _L)] for c in range(_D // _L)]
    bias_vec = bias_v[...]
    lane = lax.iota(jnp.int32, 16)

    pending = fire(0)
    meta_h.wait()

    for q in range(_NQ):
        nxt = fire(q + 1) if q + 1 < _NQ else None
        for h in pending:
            h.wait()
        pending = nxt
        ru = rubufs[q % 2]
        ri = ribufs[q % 2]

        def group(g, carry, q=q, ru=ru, ri=ri):
            res = jnp.zeros((_L,), jnp.float32)
            pv_u = poff_u[pl.ds(q * _Q + g * _L, _L)]
            pv_i = poff_i[pl.ds(q * _Q + g * _L, _L)]
            for r in range(_L):
                slot = g * _L + r
                pu = pv_u[r]
                pi = pv_i[r]
                mrow = q * (_Q // 2) + g * (_L // 2) + (r // 2)
                mcol = (r % 2) * _D
                acc = None
                for c in range(_D // _L):
                    u = ru[slot, pl.ds(pu + c * _L, _L)]
                    it = ri[slot, pl.ds(pi + c * _L, _L)]
                    m = meta_v[mrow, pl.ds(mcol + c * _L, _L)]
                    t = u * it * cw_regs[c] + m * mw_regs[c]
                    acc = t if acc is None else acc + t
                tot = jnp.sum(acc)
                res = jnp.where(lane == r, tot, res)
            out_v[pl.ds(q * _Q + g * _L, _L)] = res + bias_vec
            return carry

        lax.fori_loop(0, _Q // _L, group, 0)

    pltpu.sync_copy(out_v, out_hbm.at[pl.ds(wid * _BW, _BW)])


@jax.jit
def _run(pidx_u, pidx_i, poff_u, poff_i, meta, ut, it, cw, mw, bias16):
    mesh = plsc.VectorSubcoreMesh(core_axis_name="c", subcore_axis_name="s")
    f = functools.partial(
        pl.kernel,
        mesh=mesh,
        compiler_params=pltpu.CompilerParams(
            needs_layout_passes=False, use_tc_tiling_on_sc=True),
        out_type=jax.ShapeDtypeStruct((_B,), jnp.float32),
        scratch_types=[
            pltpu.VMEM((_NQ, _Q), jnp.int32),
            pltpu.VMEM((_NQ, _Q), jnp.int32),
            pltpu.VMEM((_BW,), jnp.int32),
            pltpu.VMEM((_BW,), jnp.int32),
            pltpu.VMEM((_Q, 2 * _D), jnp.float32),
            pltpu.VMEM((_Q, 2 * _D), jnp.float32),
            pltpu.VMEM((_Q, 2 * _D), jnp.float32),
            pltpu.VMEM((_Q, 2 * _D), jnp.float32),
            pltpu.VMEM((_BW // 2, 2 * _D), jnp.float32),
            pltpu.VMEM((_D,), jnp.float32),
            pltpu.VMEM((_D,), jnp.float32),
            pltpu.VMEM((_L,), jnp.float32),
            pltpu.VMEM((_BW,), jnp.float32),
            pltpu.SemaphoreType.DMA,
            pltpu.SemaphoreType.DMA,
        ],
    )(_sc_body)
    return f(pidx_u, pidx_i, poff_u, poff_i, meta, ut, it, cw, mw, bias16)


def kernel(user, item, item_metadata, user_table, item_table, comb_w, comb_b,
           meta_w, meta_b, global_bias):
    user = user.astype(jnp.int32)
    item = item.astype(jnp.int32)
    ut = _tc_transpose(user_table.T)
    it = _tc_transpose(item_table.T)
    hi_u = (user >= _OFF).astype(jnp.int32)
    hi_i = (item >= _OFF).astype(jnp.int32)
    pidx_u = (user - hi_u * _OFF).reshape(_NW, _NQ, _Q)
    pidx_i = (item - hi_i * _OFF).reshape(_NW, _NQ, _Q)
    poff_u = (hi_u * _D).reshape(_NW, _BW)
    poff_i = (hi_i * _D).reshape(_NW, _BW)
    meta = item_metadata.reshape(_NW, _BW // 2, 2 * _D)
    cw = comb_w.reshape(_D)
    mw = meta_w.reshape(_D)
    bias16 = jnp.broadcast_to(comb_b + meta_b + global_bias, (_L,)).astype(jnp.float32)
    return _run(pidx_u, pidx_i, poff_u, poff_i, meta, ut, it, cw, mw, bias16)


# speedup vs baseline: 4.0484x; 1.0058x over previous
import functools

import jax
import jax.numpy as jnp
from jax import lax
from jax.experimental import pallas as pl
from jax.experimental.pallas import tpu as pltpu
from jax.experimental.pallas import tpu_sc as plsc

_B = 16384
_D = 64
_L = 16

_RB = 2048
_NTB = 245
_OFF = 244 * _RB
_NPR = _NTB * _RB


def _tc_transpose_body(*refs):
    out_ref = refs[16]
    eye = jnp.eye(128, dtype=jnp.bfloat16)
    b1 = jnp.concatenate([refs[k][...] for k in range(8)], axis=0)
    b2 = jnp.concatenate([refs[8 + k][...] for k in range(8)], axis=0)
    b1 = b1.astype(jnp.bfloat16)
    b2 = b2.astype(jnp.bfloat16)
    for m in range(_RB // 128):
        sl = slice(m * 128, (m + 1) * 128)
        tr1 = jax.lax.dot_general(
            eye, b1[:, sl], (((0,), (1,)), ((), ())),
            preferred_element_type=jnp.float32)
        tr2 = jax.lax.dot_general(
            eye, b2[:, sl], (((0,), (1,)), ((), ())),
            preferred_element_type=jnp.float32)
        out_ref[pl.ds(m * 128, 128), :] = jnp.concatenate([tr1, tr2], axis=1)


def _mk_spec(k, boff):
    return pl.BlockSpec((8, _RB), lambda i, k=k, boff=boff: (k, i + boff))


def _tc_transpose(table_t):
    specs = ([_mk_spec(k, 0) for k in range(8)]
             + [_mk_spec(k, _OFF // _RB) for k in range(8)])
    return pl.pallas_call(
        _tc_transpose_body,
        grid=(_NTB,),
        in_specs=specs,
        out_specs=pl.BlockSpec((_RB, 2 * _D), lambda i: (i, 0)),
        out_shape=jax.ShapeDtypeStruct((_NPR, 2 * _D), jnp.float32),
    )(*([table_t] * 16))


_info = plsc.get_sparse_core_info()
_NC = _info.num_cores
_NS = _info.num_subcores
_NW = _NC * _NS
_BW = _B // _NW
_Q = 128
_NQ = _BW // _Q


def _sc_body(pidx_u_hbm, pidx_i_hbm, poff_u_hbm, poff_i_hbm, meta_hbm,
             ut_hbm, it_hbm, cw_hbm, mw_hbm, bias_hbm, out_hbm,
             idx_u, idx_i, poff_u, poff_i, ru0, ru1, ri0, ri1, meta_v,
             cw_v, mw_v, bias_v, out_v, sem0, sem1):
    wid = lax.axis_index("s") * _NC + lax.axis_index("c")

    pltpu.sync_copy(pidx_u_hbm.at[wid], idx_u)
    pltpu.sync_copy(pidx_i_hbm.at[wid], idx_i)
    pltpu.sync_copy(poff_u_hbm.at[wid], poff_u)
    pltpu.sync_copy(poff_i_hbm.at[wid], poff_i)
    pltpu.sync_copy(cw_hbm, cw_v)
    pltpu.sync_copy(mw_hbm, mw_v)
    pltpu.sync_copy(bias_hbm, bias_v)
    meta_h = pltpu.async_copy(meta_hbm.at[wid], meta_v, sem0)

    rubufs = (ru0, ru1)
    ribufs = (ri0, ri1)
    sems = (sem0, sem1)

    def fire(q):
        s = sems[q % 2]
        return (pltpu.async_copy(ut_hbm.at[idx_u.at[q]], rubufs[q % 2], s),
                pltpu.async_copy(it_hbm.at[idx_i.at[q]], ribufs[q % 2], s))

    cw_regs = [cw_v[pl.ds(c * _L, _L)] for c in range(_D // _L)]
    mw_regs = [mw_v[pl.ds(c * _L, _L)] for c in range(_D // _L)]
    bias_vec = bias_v[...]
    lane = lax.iota(jnp.int32, 16)

    pending = fire(0)
    meta_h.wait()

    for q in range(_NQ):
        nxt = fire(q + 1) if q + 1 < _NQ else None
        for h in pending:
            h.wait()
        pending = nxt
        ru = rubufs[q % 2]
        ri = ribufs[q % 2]

        def group(g, carry, q=q, ru=ru, ri=ri):
            res = jnp.zeros((_L,), jnp.float32)
            pv_u = poff_u[pl.ds(q * _Q + g * _L, _L)]
            pv_i = poff_i[pl.ds(q * _Q + g * _L, _L)]
            for r in range(_L):
                slot = g * _L + r
                pu = pv_u[r]
                pi = pv_i[r]
                mrow = q * (_Q // 2) + g * (_L // 2) + (r // 2)
                mcol = (r % 2) * _D
                acc = None
                for c in range(_D // _L):
                    u = ru[slot, pl.ds(pu + c * _L, _L)]
                    it = ri[slot, pl.ds(pi + c * _L, _L)]
                    m = meta_v[mrow, pl.ds(mcol + c * _L, _L)]
                    t = u * it * cw_regs[c] + m * mw_regs[c]
                    acc = t if acc is None else acc + t
                tot = jnp.sum(acc)
                res = jnp.where(lane == r, tot, res)
            out_v[pl.ds(q * _Q + g * _L, _L)] = res + bias_vec
            return carry

        lax.fori_loop(0, _Q // _L, group, 0)

    pltpu.sync_copy(out_v, out_hbm.at[pl.ds(wid * _BW, _BW)])


@jax.jit
def _run(pidx_u, pidx_i, poff_u, poff_i, meta, ut, it, cw, mw, bias16):
    mesh = plsc.VectorSubcoreMesh(core_axis_name="c", subcore_axis_name="s")
    f = functools.partial(
        pl.kernel,
        mesh=mesh,
        compiler_params=pltpu.CompilerParams(
            needs_layout_passes=False, use_tc_tiling_on_sc=True),
        out_type=jax.ShapeDtypeStruct((_B,), jnp.float32),
        scratch_types=[
            pltpu.VMEM((_NQ, _Q), jnp.int32),
            pltpu.VMEM((_NQ, _Q), jnp.int32),
            pltpu.VMEM((_BW,), jnp.int32),
            pltpu.VMEM((_BW,), jnp.int32),
            pltpu.VMEM((_Q, 2 * _D), jnp.float32),
            pltpu.VMEM((_Q, 2 * _D), jnp.float32),
            pltpu.VMEM((_Q, 2 * _D), jnp.float32),
            pltpu.VMEM((_Q, 2 * _D), jnp.float32),
            pltpu.VMEM((_BW // 2, 2 * _D), jnp.float32),
            pltpu.VMEM((_D,), jnp.float32),
            pltpu.VMEM((_D,), jnp.float32),
            pltpu.VMEM((_L,), jnp.float32),
            pltpu.VMEM((_BW,), jnp.float32),
            pltpu.SemaphoreType.DMA,
            pltpu.SemaphoreType.DMA,
        ],
    )(_sc_body)
    return f(pidx_u, pidx_i, poff_u, poff_i, meta, ut, it, cw, mw, bias16)


def kernel(user, item, item_metadata, user_table, item_table, comb_w, comb_b,
           meta_w, meta_b, global_bias):
    user = user.astype(jnp.int32)
    item = item.astype(jnp.int32)
    ut = _tc_transpose(user_table.T)
    it = _tc_transpose(item_table.T)
    hi_u = (user >= _OFF).astype(jnp.int32)
    hi_i = (item >= _OFF).astype(jnp.int32)
    pidx_u = (user - hi_u * _OFF).reshape(_NW, _NQ, _Q)
    pidx_i = (item - hi_i * _OFF).reshape(_NW, _NQ, _Q)
    poff_u = (hi_u * _D).reshape(_NW, _BW)
    poff_i = (hi_i * _D).reshape(_NW, _BW)
    meta = item_metadata.reshape(_NW, _BW // 2, 2 * _D)
    cw = comb_w.reshape(_D)
    mw = meta_w.reshape(_D)
    bias16 = jnp.broadcast_to(comb_b + meta_b + global_bias, (_L,)).astype(jnp.float32)
    return _run(pidx_u, pidx_i, poff_u, poff_i, meta, ut, it, cw, mw, bias16)


# speedup vs baseline: 5.3929x; 1.3321x over previous
import functools

import jax
import jax.numpy as jnp
from jax import lax
from jax.experimental import pallas as pl
from jax.experimental.pallas import tpu as pltpu
from jax.experimental.pallas import tpu_sc as plsc

_B = 16384
_D = 64
_L = 16

_RB = 4096
_NTB = 123
_OFF = 122 * _RB
_NPR = _NTB * _RB


def _tc_transpose_body(*refs):
    out_ref = refs[16]
    eye = jnp.eye(128, dtype=jnp.bfloat16)
    b1 = jnp.concatenate([refs[k][...] for k in range(8)], axis=0)
    b2 = jnp.concatenate([refs[8 + k][...] for k in range(8)], axis=0)
    b1 = b1.astype(jnp.bfloat16)
    b2 = b2.astype(jnp.bfloat16)
    for m in range(_RB // 128):
        sl = slice(m * 128, (m + 1) * 128)
        tr1 = jax.lax.dot_general(
            eye, b1[:, sl], (((0,), (1,)), ((), ())),
            preferred_element_type=jnp.float32)
        tr2 = jax.lax.dot_general(
            eye, b2[:, sl], (((0,), (1,)), ((), ())),
            preferred_element_type=jnp.float32)
        out_ref[pl.ds(m * 128, 128), :] = jnp.concatenate([tr1, tr2], axis=1)


def _mk_spec(k, boff):
    return pl.BlockSpec((8, _RB), lambda i, k=k, boff=boff: (k, i + boff))


def _tc_transpose(table_t):
    specs = ([_mk_spec(k, 0) for k in range(8)]
             + [_mk_spec(k, _OFF // _RB) for k in range(8)])
    return pl.pallas_call(
        _tc_transpose_body,
        grid=(_NTB,),
        in_specs=specs,
        out_specs=pl.BlockSpec((_RB, 2 * _D), lambda i: (i, 0)),
        out_shape=jax.ShapeDtypeStruct((_NPR, 2 * _D), jnp.float32),
    )(*([table_t] * 16))


_info = plsc.get_sparse_core_info()
_NC = _info.num_cores
_NS = _info.num_subcores
_NW = _NC * _NS
_BW = _B // _NW
_Q = 128
_NQ = _BW // _Q


def _sc_body(pidx_u_hbm, pidx_i_hbm, poff_u_hbm, poff_i_hbm, meta_hbm,
             ut_hbm, it_hbm, cw_hbm, mw_hbm, bias_hbm, out_hbm,
             idx_u, idx_i, poff_u, poff_i, ru0, ru1, ri0, ri1, meta_v,
             cw_v, mw_v, bias_v, out_v, sem0, sem1):
    wid = lax.axis_index("s") * _NC + lax.axis_index("c")

    pltpu.sync_copy(pidx_u_hbm.at[wid], idx_u)
    pltpu.sync_copy(pidx_i_hbm.at[wid], idx_i)
    pltpu.sync_copy(poff_u_hbm.at[wid], poff_u)
    pltpu.sync_copy(poff_i_hbm.at[wid], poff_i)
    pltpu.sync_copy(cw_hbm, cw_v)
    pltpu.sync_copy(mw_hbm, mw_v)
    pltpu.sync_copy(bias_hbm, bias_v)
    meta_h = pltpu.async_copy(meta_hbm.at[wid], meta_v, sem0)

    rubufs = (ru0, ru1)
    ribufs = (ri0, ri1)
    sems = (sem0, sem1)

    def fire(q):
        s = sems[q % 2]
        return (pltpu.async_copy(ut_hbm.at[idx_u.at[q]], rubufs[q % 2], s),
                pltpu.async_copy(it_hbm.at[idx_i.at[q]], ribufs[q % 2], s))

    cw_regs = [cw_v[pl.ds(c * _L, _L)] for c in range(_D // _L)]
    mw_regs = [mw_v[pl.ds(c * _L, _L)] for c in range(_D // _L)]
    bias_vec = bias_v[...]
    lane = lax.iota(jnp.int32, 16)

    pending = fire(0)
    meta_h.wait()

    for q in range(_NQ):
        nxt = fire(q + 1) if q + 1 < _NQ else None
        for h in pending:
            h.wait()
        pending = nxt
        ru = rubufs[q % 2]
        ri = ribufs[q % 2]

        def group(g, carry, q=q, ru=ru, ri=ri):
            res = jnp.zeros((_L,), jnp.float32)
            pv_u = poff_u[pl.ds(q * _Q + g * _L, _L)]
            pv_i = poff_i[pl.ds(q * _Q + g * _L, _L)]
            for r in range(_L):
                slot = g * _L + r
                pu = pv_u[r]
                pi = pv_i[r]
                mrow = q * (_Q // 2) + g * (_L // 2) + (r // 2)
                mcol = (r % 2) * _D
                acc = None
                for c in range(_D // _L):
                    u = ru[slot, pl.ds(pu + c * _L, _L)]
                    it = ri[slot, pl.ds(pi + c * _L, _L)]
                    m = meta_v[mrow, pl.ds(mcol + c * _L, _L)]
                    t = u * it * cw_regs[c] + m * mw_regs[c]
                    acc = t if acc is None else acc + t
                tot = jnp.sum(acc)
                res = jnp.where(lane == r, tot, res)
            out_v[pl.ds(q * _Q + g * _L, _L)] = res + bias_vec
            return carry

        lax.fori_loop(0, _Q // _L, group, 0)

    pltpu.sync_copy(out_v, out_hbm.at[pl.ds(wid * _BW, _BW)])


@jax.jit
def _run(pidx_u, pidx_i, poff_u, poff_i, meta, ut, it, cw, mw, bias16):
    mesh = plsc.VectorSubcoreMesh(core_axis_name="c", subcore_axis_name="s")
    f = functools.partial(
        pl.kernel,
        mesh=mesh,
        compiler_params=pltpu.CompilerParams(
            needs_layout_passes=False, use_tc_tiling_on_sc=True),
        out_type=jax.ShapeDtypeStruct((_B,), jnp.float32),
        scratch_types=[
            pltpu.VMEM((_NQ, _Q), jnp.int32),
            pltpu.VMEM((_NQ, _Q), jnp.int32),
            pltpu.VMEM((_BW,), jnp.int32),
            pltpu.VMEM((_BW,), jnp.int32),
            pltpu.VMEM((_Q, 2 * _D), jnp.float32),
            pltpu.VMEM((_Q, 2 * _D), jnp.float32),
            pltpu.VMEM((_Q, 2 * _D), jnp.float32),
            pltpu.VMEM((_Q, 2 * _D), jnp.float32),
            pltpu.VMEM((_BW // 2, 2 * _D), jnp.float32),
            pltpu.VMEM((_D,), jnp.float32),
            pltpu.VMEM((_D,), jnp.float32),
            pltpu.VMEM((_L,), jnp.float32),
            pltpu.VMEM((_BW,), jnp.float32),
            pltpu.SemaphoreType.DMA,
            pltpu.SemaphoreType.DMA,
        ],
    )(_sc_body)
    return f(pidx_u, pidx_i, poff_u, poff_i, meta, ut, it, cw, mw, bias16)


def kernel(user, item, item_metadata, user_table, item_table, comb_w, comb_b,
           meta_w, meta_b, global_bias):
    user = user.astype(jnp.int32)
    item = item.astype(jnp.int32)
    ut = _tc_transpose(user_table.T)
    it = _tc_transpose(item_table.T)
    hi_u = (user >= _OFF).astype(jnp.int32)
    hi_i = (item >= _OFF).astype(jnp.int32)
    pidx_u = (user - hi_u * _OFF).reshape(_NW, _NQ, _Q)
    pidx_i = (item - hi_i * _OFF).reshape(_NW, _NQ, _Q)
    poff_u = (hi_u * _D).reshape(_NW, _BW)
    poff_i = (hi_i * _D).reshape(_NW, _BW)
    meta = item_metadata.reshape(_NW, _BW // 2, 2 * _D)
    cw = comb_w.reshape(_D)
    mw = meta_w.reshape(_D)
    bias16 = jnp.broadcast_to(comb_b + meta_b + global_bias, (_L,)).astype(jnp.float32)
    return _run(pidx_u, pidx_i, poff_u, poff_i, meta, ut, it, cw, mw, bias16)


# speedup vs baseline: 6.4072x; 1.1881x over previous
import functools

import jax
import jax.numpy as jnp
from jax import lax
from jax.experimental import pallas as pl
from jax.experimental.pallas import tpu as pltpu
from jax.experimental.pallas import tpu_sc as plsc

_B = 16384
_D = 64
_L = 16

_RB = 8192
_NTB = 62
_OFF = 61 * _RB
_NPR = _NTB * _RB


def _tc_transpose_body(*refs):
    out_ref = refs[16]
    eye = jnp.eye(128, dtype=jnp.bfloat16)
    b1 = jnp.concatenate([refs[k][...] for k in range(8)], axis=0)
    b2 = jnp.concatenate([refs[8 + k][...] for k in range(8)], axis=0)
    b1 = b1.astype(jnp.bfloat16)
    b2 = b2.astype(jnp.bfloat16)
    for m in range(_RB // 128):
        sl = slice(m * 128, (m + 1) * 128)
        tr1 = jax.lax.dot_general(
            eye, b1[:, sl], (((0,), (1,)), ((), ())),
            preferred_element_type=jnp.float32)
        tr2 = jax.lax.dot_general(
            eye, b2[:, sl], (((0,), (1,)), ((), ())),
            preferred_element_type=jnp.float32)
        out_ref[pl.ds(m * 128, 128), :] = jnp.concatenate([tr1, tr2], axis=1)


def _mk_spec(k, boff):
    return pl.BlockSpec((8, _RB), lambda i, k=k, boff=boff: (k, i + boff))


def _tc_transpose(table_t):
    specs = ([_mk_spec(k, 0) for k in range(8)]
             + [_mk_spec(k, _OFF // _RB) for k in range(8)])
    return pl.pallas_call(
        _tc_transpose_body,
        grid=(_NTB,),
        in_specs=specs,
        out_specs=pl.BlockSpec((_RB, 2 * _D), lambda i: (i, 0)),
        out_shape=jax.ShapeDtypeStruct((_NPR, 2 * _D), jnp.float32),
    )(*([table_t] * 16))


_info = plsc.get_sparse_core_info()
_NC = _info.num_cores
_NS = _info.num_subcores
_NW = _NC * _NS
_BW = _B // _NW
_Q = 128
_NQ = _BW // _Q


def _sc_body(pidx_u_hbm, pidx_i_hbm, poff_u_hbm, poff_i_hbm, meta_hbm,
             ut_hbm, it_hbm, cw_hbm, mw_hbm, bias_hbm, out_hbm,
             idx_u, idx_i, poff_u, poff_i, ru0, ru1, ri0, ri1, meta_v,
             cw_v, mw_v, bias_v, out_v, sem0, sem1):
    wid = lax.axis_index("s") * _NC + lax.axis_index("c")

    pltpu.sync_copy(pidx_u_hbm.at[wid], idx_u)
    pltpu.sync_copy(pidx_i_hbm.at[wid], idx_i)
    pltpu.sync_copy(poff_u_hbm.at[wid], poff_u)
    pltpu.sync_copy(poff_i_hbm.at[wid], poff_i)
    pltpu.sync_copy(cw_hbm, cw_v)
    pltpu.sync_copy(mw_hbm, mw_v)
    pltpu.sync_copy(bias_hbm, bias_v)
    meta_h = pltpu.async_copy(meta_hbm.at[wid], meta_v, sem0)

    rubufs = (ru0, ru1)
    ribufs = (ri0, ri1)
    sems = (sem0, sem1)

    def fire(q):
        s = sems[q % 2]
        return (pltpu.async_copy(ut_hbm.at[idx_u.at[q]], rubufs[q % 2], s),
                pltpu.async_copy(it_hbm.at[idx_i.at[q]], ribufs[q % 2], s))

    cw_regs = [cw_v[pl.ds(c * _L, _L)] for c in range(_D // _L)]
    mw_regs = [mw_v[pl.ds(c * _L, _L)] for c in range(_D // _L)]
    bias_vec = bias_v[...]
    lane = lax.iota(jnp.int32, 16)

    pending = fire(0)
    meta_h.wait()

    for q in range(_NQ):
        nxt = fire(q + 1) if q + 1 < _NQ else None
        for h in pending:
            h.wait()
        pending = nxt
        ru = rubufs[q % 2]
        ri = ribufs[q % 2]

        def group(g, carry, q=q, ru=ru, ri=ri):
            res = jnp.zeros((_L,), jnp.float32)
            pv_u = poff_u[pl.ds(q * _Q + g * _L, _L)]
            pv_i = poff_i[pl.ds(q * _Q + g * _L, _L)]
            for r in range(_L):
                slot = g * _L + r
                pu = pv_u[r]
                pi = pv_i[r]
                mrow = q * (_Q // 2) + g * (_L // 2) + (r // 2)
                mcol = (r % 2) * _D
                acc = None
                for c in range(_D // _L):
                    u = ru[slot, pl.ds(pu + c * _L, _L)]
                    it = ri[slot, pl.ds(pi + c * _L, _L)]
                    m = meta_v[mrow, pl.ds(mcol + c * _L, _L)]
                    t = u * it * cw_regs[c] + m * mw_regs[c]
                    acc = t if acc is None else acc + t
                tot = jnp.sum(acc)
                res = jnp.where(lane == r, tot, res)
            out_v[pl.ds(q * _Q + g * _L, _L)] = res + bias_vec
            return carry

        lax.fori_loop(0, _Q // _L, group, 0)

    pltpu.sync_copy(out_v, out_hbm.at[pl.ds(wid * _BW, _BW)])


@jax.jit
def _run(pidx_u, pidx_i, poff_u, poff_i, meta, ut, it, cw, mw, bias16):
    mesh = plsc.VectorSubcoreMesh(core_axis_name="c", subcore_axis_name="s")
    f = functools.partial(
        pl.kernel,
        mesh=mesh,
        compiler_params=pltpu.CompilerParams(
            needs_layout_passes=False, use_tc_tiling_on_sc=True),
        out_type=jax.ShapeDtypeStruct((_B,), jnp.float32),
        scratch_types=[
            pltpu.VMEM((_NQ, _Q), jnp.int32),
            pltpu.VMEM((_NQ, _Q), jnp.int32),
            pltpu.VMEM((_BW,), jnp.int32),
            pltpu.VMEM((_BW,), jnp.int32),
            pltpu.VMEM((_Q, 2 * _D), jnp.float32),
            pltpu.VMEM((_Q, 2 * _D), jnp.float32),
            pltpu.VMEM((_Q, 2 * _D), jnp.float32),
            pltpu.VMEM((_Q, 2 * _D), jnp.float32),
            pltpu.VMEM((_BW // 2, 2 * _D), jnp.float32),
            pltpu.VMEM((_D,), jnp.float32),
            pltpu.VMEM((_D,), jnp.float32),
            pltpu.VMEM((_L,), jnp.float32),
            pltpu.VMEM((_BW,), jnp.float32),
            pltpu.SemaphoreType.DMA,
            pltpu.SemaphoreType.DMA,
        ],
    )(_sc_body)
    return f(pidx_u, pidx_i, poff_u, poff_i, meta, ut, it, cw, mw, bias16)


def kernel(user, item, item_metadata, user_table, item_table, comb_w, comb_b,
           meta_w, meta_b, global_bias):
    user = user.astype(jnp.int32)
    item = item.astype(jnp.int32)
    ut = _tc_transpose(user_table.T)
    it = _tc_transpose(item_table.T)
    hi_u = (user >= _OFF).astype(jnp.int32)
    hi_i = (item >= _OFF).astype(jnp.int32)
    pidx_u = (user - hi_u * _OFF).reshape(_NW, _NQ, _Q)
    pidx_i = (item - hi_i * _OFF).reshape(_NW, _NQ, _Q)
    poff_u = (hi_u * _D).reshape(_NW, _BW)
    poff_i = (hi_i * _D).reshape(_NW, _BW)
    meta = item_metadata.reshape(_NW, _BW // 2, 2 * _D)
    cw = comb_w.reshape(_D)
    mw = meta_w.reshape(_D)
    bias16 = jnp.broadcast_to(comb_b + meta_b + global_bias, (_L,)).astype(jnp.float32)
    return _run(pidx_u, pidx_i, poff_u, poff_i, meta, ut, it, cw, mw, bias16)


# speedup vs baseline: 6.4965x; 1.0139x over previous
import functools

import jax
import jax.numpy as jnp
from jax import lax
from jax.experimental import pallas as pl
from jax.experimental.pallas import tpu as pltpu
from jax.experimental.pallas import tpu_sc as plsc

_B = 16384
_D = 64
_L = 16

_RB = 16384
_NTB = 32
_OFF = 30 * _RB
_NPR = _NTB * _RB


def _tc_transpose_body(*refs):
    out_ref = refs[16]
    eye = jnp.eye(128, dtype=jnp.bfloat16)
    b1 = jnp.concatenate([refs[k][...] for k in range(8)], axis=0)
    b2 = jnp.concatenate([refs[8 + k][...] for k in range(8)], axis=0)
    b1 = b1.astype(jnp.bfloat16)
    b2 = b2.astype(jnp.bfloat16)
    for m in range(_RB // 128):
        sl = slice(m * 128, (m + 1) * 128)
        tr1 = jax.lax.dot_general(
            eye, b1[:, sl], (((0,), (1,)), ((), ())),
            preferred_element_type=jnp.float32)
        tr2 = jax.lax.dot_general(
            eye, b2[:, sl], (((0,), (1,)), ((), ())),
            preferred_element_type=jnp.float32)
        out_ref[pl.ds(m * 128, 128), :] = jnp.concatenate([tr1, tr2], axis=1)


def _mk_spec(k, boff):
    return pl.BlockSpec((8, _RB), lambda i, k=k, boff=boff: (k, i + boff))


def _tc_transpose(table_t):
    specs = ([_mk_spec(k, 0) for k in range(8)]
             + [_mk_spec(k, _OFF // _RB) for k in range(8)])
    return pl.pallas_call(
        _tc_transpose_body,
        grid=(_NTB,),
        in_specs=specs,
        out_specs=pl.BlockSpec((_RB, 2 * _D), lambda i: (i, 0)),
        out_shape=jax.ShapeDtypeStruct((_NPR, 2 * _D), jnp.float32),
    )(*([table_t] * 16))


_info = plsc.get_sparse_core_info()
_NC = _info.num_cores
_NS = _info.num_subcores
_NW = _NC * _NS
_BW = _B // _NW
_Q = 128
_NQ = _BW // _Q


def _sc_body(pidx_u_hbm, pidx_i_hbm, poff_u_hbm, poff_i_hbm, meta_hbm,
             ut_hbm, it_hbm, cw_hbm, mw_hbm, bias_hbm, out_hbm,
             idx_u, idx_i, poff_u, poff_i, ru0, ru1, ri0, ri1, meta_v,
             cw_v, mw_v, bias_v, out_v, sem0, sem1):
    wid = lax.axis_index("s") * _NC + lax.axis_index("c")

    pltpu.sync_copy(pidx_u_hbm.at[wid], idx_u)
    pltpu.sync_copy(pidx_i_hbm.at[wid], idx_i)
    pltpu.sync_copy(poff_u_hbm.at[wid], poff_u)
    pltpu.sync_copy(poff_i_hbm.at[wid], poff_i)
    pltpu.sync_copy(cw_hbm, cw_v)
    pltpu.sync_copy(mw_hbm, mw_v)
    pltpu.sync_copy(bias_hbm, bias_v)
    meta_h = pltpu.async_copy(meta_hbm.at[wid], meta_v, sem0)

    rubufs = (ru0, ru1)
    ribufs = (ri0, ri1)
    sems = (sem0, sem1)

    def fire(q):
        s = sems[q % 2]
        return (pltpu.async_copy(ut_hbm.at[idx_u.at[q]], rubufs[q % 2], s),
                pltpu.async_copy(it_hbm.at[idx_i.at[q]], ribufs[q % 2], s))

    cw_regs = [cw_v[pl.ds(c * _L, _L)] for c in range(_D // _L)]
    mw_regs = [mw_v[pl.ds(c * _L, _L)] for c in range(_D // _L)]
    bias_vec = bias_v[...]
    lane = lax.iota(jnp.int32, 16)

    pending = fire(0)
    meta_h.wait()

    for q in range(_NQ):
        nxt = fire(q + 1) if q + 1 < _NQ else None
        for h in pending:
            h.wait()
        pending = nxt
        ru = rubufs[q % 2]
        ri = ribufs[q % 2]

        def group(g, carry, q=q, ru=ru, ri=ri):
            res = jnp.zeros((_L,), jnp.float32)
            pv_u = poff_u[pl.ds(q * _Q + g * _L, _L)]
            pv_i = poff_i[pl.ds(q * _Q + g * _L, _L)]
            for r in range(_L):
                slot = g * _L + r
                pu = pv_u[r]
                pi = pv_i[r]
                mrow = q * (_Q // 2) + g * (_L // 2) + (r // 2)
                mcol = (r % 2) * _D
                acc = None
                for c in range(_D // _L):
                    u = ru[slot, pl.ds(pu + c * _L, _L)]
                    it = ri[slot, pl.ds(pi + c * _L, _L)]
                    m = meta_v[mrow, pl.ds(mcol + c * _L, _L)]
                    t = u * it * cw_regs[c] + m * mw_regs[c]
                    acc = t if acc is None else acc + t
                tot = jnp.sum(acc)
                res = jnp.where(lane == r, tot, res)
            out_v[pl.ds(q * _Q + g * _L, _L)] = res + bias_vec
            return carry

        lax.fori_loop(0, _Q // _L, group, 0)

    pltpu.sync_copy(out_v, out_hbm.at[pl.ds(wid * _BW, _BW)])


@jax.jit
def _run(pidx_u, pidx_i, poff_u, poff_i, meta, ut, it, cw, mw, bias16):
    mesh = plsc.VectorSubcoreMesh(core_axis_name="c", subcore_axis_name="s")
    f = functools.partial(
        pl.kernel,
        mesh=mesh,
        compiler_params=pltpu.CompilerParams(
            needs_layout_passes=False, use_tc_tiling_on_sc=True),
        out_type=jax.ShapeDtypeStruct((_B,), jnp.float32),
        scratch_types=[
            pltpu.VMEM((_NQ, _Q), jnp.int32),
            pltpu.VMEM((_NQ, _Q), jnp.int32),
            pltpu.VMEM((_BW,), jnp.int32),
            pltpu.VMEM((_BW,), jnp.int32),
            pltpu.VMEM((_Q, 2 * _D), jnp.float32),
            pltpu.VMEM((_Q, 2 * _D), jnp.float32),
            pltpu.VMEM((_Q, 2 * _D), jnp.float32),
            pltpu.VMEM((_Q, 2 * _D), jnp.float32),
            pltpu.VMEM((_BW // 2, 2 * _D), jnp.float32),
            pltpu.VMEM((_D,), jnp.float32),
            pltpu.VMEM((_D,), jnp.float32),
            pltpu.VMEM((_L,), jnp.float32),
            pltpu.VMEM((_BW,), jnp.float32),
            pltpu.SemaphoreType.DMA,
            pltpu.SemaphoreType.DMA,
        ],
    )(_sc_body)
    return f(pidx_u, pidx_i, poff_u, poff_i, meta, ut, it, cw, mw, bias16)


def kernel(user, item, item_metadata, user_table, item_table, comb_w, comb_b,
           meta_w, meta_b, global_bias):
    user = user.astype(jnp.int32)
    item = item.astype(jnp.int32)
    ut = _tc_transpose(user_table.T)
    it = _tc_transpose(item_table.T)
    hi_u = (user >= _OFF).astype(jnp.int32)
    hi_i = (item >= _OFF).astype(jnp.int32)
    pidx_u = (user - hi_u * _OFF).reshape(_NW, _NQ, _Q)
    pidx_i = (item - hi_i * _OFF).reshape(_NW, _NQ, _Q)
    poff_u = (hi_u * _D).reshape(_NW, _BW)
    poff_i = (hi_i * _D).reshape(_NW, _BW)
    meta = item_metadata.reshape(_NW, _BW // 2, 2 * _D)
    cw = comb_w.reshape(_D)
    mw = meta_w.reshape(_D)
    bias16 = jnp.broadcast_to(comb_b + meta_b + global_bias, (_L,)).astype(jnp.float32)
    return _run(pidx_u, pidx_i, poff_u, poff_i, meta, ut, it, cw, mw, bias16)
